# baseline (speedup 1.0000x reference)
import functools

import jax
import jax.numpy as jnp
from jax import lax
from jax.experimental import pallas as pl
from jax.experimental.pallas import tpu as pltpu
from jax.experimental.pallas import tpu_sc as plsc

NC = 2
NS = 16
NW = NC * NS
DG = 16


def _make_seg_sum(n, e, d, ch, nbuf, with_deg):
  et = e // NW
  nchunks = et // ch
  nslot = 2 * nbuf
  ngroups = nchunks // nslot
  ntail = nchunks % nslot
  assert et % ch == 0 and ch % 8 == 0 and nchunks >= nslot
  rpt = (n // NS + 7) // 8 * 8
  npad = rpt * NS

  out_type = [jax.ShapeDtypeStruct((NC, npad, d), jnp.float32)]
  scratch = [
      pltpu.VMEM((nslot, 2, ch), jnp.int32),
      pltpu.VMEM((nbuf, ch, d), jnp.float32),
      pltpu.VMEM_SHARED((npad, d), jnp.float32),
      pltpu.SemaphoreType.DMA((nslot,)),
      pltpu.SemaphoreType.DMA((nslot,)),
      pltpu.SemaphoreType.DMA((nbuf,)),
  ]
  if with_deg:
    out_type.append(jax.ShapeDtypeStruct((NC, npad, DG), jnp.float32))
    scratch.append(pltpu.VMEM((ch, DG), jnp.float32))
    scratch.append(pltpu.VMEM_SHARED((npad, DG), jnp.float32))

  mesh = plsc.VectorSubcoreMesh(core_axis_name="c", subcore_axis_name="s")

  def body(tab_hbm, ei_hbm, zrows_hbm, zdeg_hbm, out_hbm, deg_hbm,
           idx_v, rows_v, table_s, isems_s, isems_d, gsems,
           ones_v, degtab_s):
    c = lax.axis_index("c")
    s = lax.axis_index("s")
    wid = s * NC + c
    ebase = wid * et

    pltpu.sync_copy(zrows_hbm, table_s.at[pl.ds(s * rpt, rpt)])
    if with_deg:
      pltpu.sync_copy(zdeg_hbm, degtab_s.at[pl.ds(s * rpt, rpt)])
      for i in range(ch):
        ones_v[i] = jnp.ones((DG,), jnp.float32)

    def load_idx(j, q):
      eb = ebase + j * ch
      pltpu.async_copy(ei_hbm.at[0, pl.ds(eb, ch)], idx_v.at[q, 0],
                       isems_s.at[q])
      pltpu.async_copy(ei_hbm.at[1, pl.ds(eb, ch)], idx_v.at[q, 1],
                       isems_d.at[q])

    def issue_gather(q, b):
      pltpu.make_async_copy(ei_hbm.at[0, pl.ds(0, ch)], idx_v.at[q, 0],
                            isems_s.at[q]).wait()
      pltpu.async_copy(tab_hbm.at[idx_v.at[q, 0]], rows_v.at[b], gsems.at[b])

    for q in range(nslot):
      load_idx(q, q)
    for b in range(nbuf):
      issue_gather(b, b)

    plsc.subcore_barrier()

    def stage(j, u, guard):
      b = u % nbuf
      pltpu.make_async_copy(tab_hbm.at[pl.ds(0, ch)], rows_v.at[b],
                            gsems.at[b]).wait()
      pltpu.make_async_copy(ei_hbm.at[0, pl.ds(0, ch)], idx_v.at[u, 1],
                            isems_d.at[u]).wait()
      pltpu.sync_copy(rows_v.at[b], table_s.at[idx_v.at[u, 1]], add=True)
      if with_deg:
        pltpu.sync_copy(ones_v, degtab_s.at[idx_v.at[u, 1]], add=True)

      guard(j + nslot < nchunks, lambda: load_idx(j + nslot, u))
      qn = (u + nbuf) % nslot
      guard(j + nbuf < nchunks, lambda: issue_gather(qn, b))

    def traced_guard(cond, fn):
      pl.when(cond)(fn)

    def static_guard(cond, fn):
      if cond:
        fn()

    def group(g, carry):
      for u in range(nslot):
        stage(g * nslot + u, u, traced_guard)
      return carry

    lax.fori_loop(0, ngroups, group, 0, unroll=False)
    for u in range(ntail):
      stage(ngroups * nslot + u, u, static_guard)
    plsc.subcore_barrier()

    pltpu.sync_copy(table_s.at[pl.ds(s * rpt, rpt)],
                    out_hbm.at[c, pl.ds(s * rpt, rpt)])
    if with_deg:
      pltpu.sync_copy(degtab_s.at[pl.ds(s * rpt, rpt)],
                      deg_hbm.at[c, pl.ds(s * rpt, rpt)])

  kern = functools.partial(
      pl.kernel,
      mesh=mesh,
      compiler_params=pltpu.CompilerParams(use_tc_tiling_on_sc=False),
      out_type=tuple(out_type) if with_deg else out_type[0],
      scratch_types=scratch,
  )

  if with_deg:
    @kern
    def seg(tab, ei, zrows, zdeg, out, deg,
            idx_v, rows_v, table_s, isems_s, isems_d, gsems, ones_v, degtab_s):
      body(tab, ei, zrows, zdeg, out, deg,
           idx_v, rows_v, table_s, isems_s, isems_d, gsems, ones_v, degtab_s)
  else:
    @kern
    def seg(tab, ei, zrows, out,
            idx_v, rows_v, table_s, isems_s, isems_d, gsems):
      body(tab, ei, zrows, None, out, None,
           idx_v, rows_v, table_s, isems_s, isems_d, gsems, None, None)

  return seg


def _flat_to_rows(flat):
  fr = flat.shape[0]
  rows = fr * 8
  i0 = lax.broadcasted_iota(jnp.int32, (rows, fr), 0) // 8
  k0 = lax.broadcasted_iota(jnp.int32, (rows, fr), 1)
  expand = jnp.where(i0 == k0, 1.0, 0.0).astype(jnp.float32)
  ex = jnp.dot(expand, flat, preferred_element_type=jnp.float32)
  rmod = lax.broadcasted_iota(jnp.int32, (rows, DG), 0) % 8
  out = jnp.zeros((rows, DG), jnp.float32)
  for g in range(8):
    out = out + jnp.where(rmod == g, ex[:, g * DG:(g + 1) * DG], 0.0)
  return out


def _rows_to_flat(p):
  rows = p.shape[0]
  fr = rows // 8
  pieces = []
  r0 = lax.broadcasted_iota(jnp.int32, (fr, rows), 0)
  c0 = lax.broadcasted_iota(jnp.int32, (fr, rows), 1)
  for g in range(8):
    sel = jnp.where(c0 == 8 * r0 + g, 1.0, 0.0).astype(jnp.float32)
    pieces.append(jnp.dot(sel, p, preferred_element_type=jnp.float32))
  return jnp.concatenate(pieces, axis=1)


def _inv_deg(degt_ref):
  deg16 = _flat_to_rows(degt_ref[0] + degt_ref[1])
  return 1.0 / (jnp.max(deg16, axis=1, keepdims=True) + 1.0)


def _layer1_body(feat_ref, degt_ref, x_ref, w1_ref, b1_ref, w2_ref, p_ref):
  aggx = feat_ref[0] + feat_ref[1] + x_ref[...]
  h = aggx * _inv_deg(degt_ref)
  h1 = jnp.maximum(
      jnp.dot(h, w1_ref[...], preferred_element_type=jnp.float32)
      + b1_ref[...], 0.0)
  p = jnp.dot(h1, w2_ref[...], preferred_element_type=jnp.float32)
  p_ref[...] = _rows_to_flat(p)


def _layer2_body(agg_ref, p_ref, degt_ref, b2_ref, z_ref):
  ap = _flat_to_rows(agg_ref[0] + agg_ref[1] + p_ref[...])
  z_ref[...] = ap * _inv_deg(degt_ref) + b2_ref[...]


def _decoder_body(zr_ref, zc_ref, out_ref):
  out_ref[...] = lax.dot_general(
      zr_ref[...], zc_ref[...], (((1,), (1,)), ((), ())),
      preferred_element_type=jnp.float32)


def kernel(x, edge_index, W1, b1, W2, b2):
  n, din = x.shape
  e = edge_index.shape[1]
  h1_dim = W1.shape[1]
  h2 = W2.shape[1]

  rpt = (n // NS + 7) // 8 * 8
  z1 = jnp.zeros((rpt, din), jnp.float32)
  zd = jnp.zeros((rpt, DG), jnp.float32)
  z2 = jnp.zeros((rpt, h2), jnp.float32)

  seg1 = _make_seg_sum(n, e, din, 80, 3, with_deg=True)
  seg2 = _make_seg_sum(n, e, h2, 80, 3, with_deg=False)

  feat, degt = seg1(x, edge_index, z1, zd)
  npad = rpt * NS
  degt_f = degt.reshape(NC, npad * DG // 128, 128)

  rblk = 1024
  fblk = rblk * h2 // 128
  dblk = rblk * DG // 128
  grid1 = pl.cdiv(n, rblk)
  p_f = pl.pallas_call(
      _layer1_body,
      grid=(grid1,),
      in_specs=[
          pl.BlockSpec((NC, rblk, din), lambda i: (0, i, 0)),
          pl.BlockSpec((NC, dblk, 128), lambda i: (0, i, 0)),
          pl.BlockSpec((rblk, din), lambda i: (i, 0)),
          pl.BlockSpec((din, h1_dim), lambda i: (0, 0)),
          pl.BlockSpec((1, h1_dim), lambda i: (0, 0)),
          pl.BlockSpec((h1_dim, h2), lambda i: (0, 0)),
      ],
      out_specs=pl.BlockSpec((fblk, 128), lambda i: (i, 0)),
      out_shape=jax.ShapeDtypeStruct((n * h2 // 128, 128), jnp.float32),
  )(feat, degt_f, x, W1, b1.reshape(1, h1_dim), W2)

  agg2 = seg2(p_f.reshape(n, h2), edge_index, z2)
  agg2_f = agg2.reshape(NC, npad * h2 // 128, 128)

  z = pl.pallas_call(
      _layer2_body,
      grid=(grid1,),
      in_specs=[
          pl.BlockSpec((NC, fblk, 128), lambda i: (0, i, 0)),
          pl.BlockSpec((fblk, 128), lambda i: (i, 0)),
          pl.BlockSpec((NC, dblk, 128), lambda i: (0, i, 0)),
          pl.BlockSpec((1, h2), lambda i: (0, 0)),
      ],
      out_specs=pl.BlockSpec((rblk, h2), lambda i: (i, 0)),
      out_shape=jax.ShapeDtypeStruct((n, h2), jnp.float32),
  )(agg2_f, p_f, degt_f, b2.reshape(1, h2))

  ablk = 1024
  grid_a = pl.cdiv(n, ablk)
  adj = pl.pallas_call(
      _decoder_body,
      grid=(grid_a, grid_a),
      in_specs=[
          pl.BlockSpec((ablk, h2), lambda i, j: (i, 0)),
          pl.BlockSpec((ablk, h2), lambda i, j: (j, 0)),
      ],
      out_specs=pl.BlockSpec((ablk, ablk), lambda i, j: (i, j)),
      out_shape=jax.ShapeDtypeStruct((n, n), jnp.float32),
  )(z, z)

  return (z, adj)


# speedup vs baseline: 1.0916x; 1.0916x over previous
import functools

import jax
import jax.numpy as jnp
from jax import lax
from jax.experimental import pallas as pl
from jax.experimental.pallas import tpu as pltpu
from jax.experimental.pallas import tpu_sc as plsc

NC = 2
NS = 16
NW = NC * NS
DG = 16


def _make_seg_sum(n, e, d, ch, nbuf, with_deg):
  et = e // NW
  nchunks = et // ch
  etail = et % ch
  nslot = 2 * nbuf
  ngroups = nchunks // nslot
  ntail = nchunks % nslot
  assert ch % 8 == 0 and (nchunks * ch) % 8 == 0 and nchunks >= nslot
  rpt = (n // NS + 7) // 8 * 8
  npad = rpt * NS

  out_type = [jax.ShapeDtypeStruct((NC, npad, d), jnp.float32)]
  scratch = [
      pltpu.VMEM((nslot, 2, ch), jnp.int32),
      pltpu.VMEM((nbuf, ch, d), jnp.float32),
      pltpu.VMEM_SHARED((npad, d), jnp.float32),
      pltpu.SemaphoreType.DMA((nslot,)),
      pltpu.SemaphoreType.DMA((nslot,)),
      pltpu.SemaphoreType.DMA((nbuf,)),
  ]
  if with_deg:
    out_type.append(jax.ShapeDtypeStruct((NC, npad, DG), jnp.float32))
    scratch.append(pltpu.VMEM((ch, DG), jnp.float32))
    scratch.append(pltpu.VMEM_SHARED((npad, DG), jnp.float32))
  if etail:
    scratch.append(pltpu.VMEM((2, etail), jnp.int32))
    scratch.append(pltpu.VMEM((etail, d), jnp.float32))
    scratch.append(pltpu.SemaphoreType.DMA)

  mesh = plsc.VectorSubcoreMesh(core_axis_name="c", subcore_axis_name="s")

  def body(tab_hbm, ei_hbm, zrows_hbm, zdeg_hbm, out_hbm, deg_hbm,
           idx_v, rows_v, table_s, isems_s, isems_d, gsems,
           ones_v, degtab_s, tidx_v, trows_v, tsem):
    c = lax.axis_index("c")
    s = lax.axis_index("s")
    wid = s * NC + c
    ebase = wid * et

    pltpu.sync_copy(zrows_hbm, table_s.at[pl.ds(s * rpt, rpt)])
    if with_deg:
      pltpu.sync_copy(zdeg_hbm, degtab_s.at[pl.ds(s * rpt, rpt)])
      for i in range(ch):
        ones_v[i] = jnp.ones((DG,), jnp.float32)

    def load_idx(j, q):
      eb = ebase + j * ch
      pltpu.async_copy(ei_hbm.at[0, pl.ds(eb, ch)], idx_v.at[q, 0],
                       isems_s.at[q])
      pltpu.async_copy(ei_hbm.at[1, pl.ds(eb, ch)], idx_v.at[q, 1],
                       isems_d.at[q])

    def issue_gather(q, b):
      pltpu.make_async_copy(ei_hbm.at[0, pl.ds(0, ch)], idx_v.at[q, 0],
                            isems_s.at[q]).wait()
      pltpu.async_copy(tab_hbm.at[idx_v.at[q, 0]], rows_v.at[b], gsems.at[b])

    for q in range(nslot):
      load_idx(q, q)
    for b in range(nbuf):
      issue_gather(b, b)

    plsc.subcore_barrier()

    if etail:
      tb = ebase + nchunks * ch
      pltpu.sync_copy(ei_hbm.at[0, pl.ds(tb, etail)], tidx_v.at[0])
      pltpu.sync_copy(ei_hbm.at[1, pl.ds(tb, etail)], tidx_v.at[1])
      pltpu.async_copy(tab_hbm.at[tidx_v.at[0]], trows_v, tsem).wait()
      pltpu.sync_copy(trows_v, table_s.at[tidx_v.at[1]], add=True)
      if with_deg:
        pltpu.sync_copy(ones_v.at[pl.ds(0, etail)],
                        degtab_s.at[tidx_v.at[1]], add=True)

    def stage(j, u, guard):
      b = u % nbuf
      pltpu.make_async_copy(tab_hbm.at[pl.ds(0, ch)], rows_v.at[b],
                            gsems.at[b]).wait()
      pltpu.make_async_copy(ei_hbm.at[0, pl.ds(0, ch)], idx_v.at[u, 1],
                            isems_d.at[u]).wait()
      pltpu.sync_copy(rows_v.at[b], table_s.at[idx_v.at[u, 1]], add=True)
      if with_deg:
        pltpu.sync_copy(ones_v, degtab_s.at[idx_v.at[u, 1]], add=True)

      guard(j + nslot < nchunks, lambda: load_idx(j + nslot, u))
      qn = (u + nbuf) % nslot
      guard(j + nbuf < nchunks, lambda: issue_gather(qn, b))

    def traced_guard(cond, fn):
      pl.when(cond)(fn)

    def static_guard(cond, fn):
      if cond:
        fn()

    def group(g, carry):
      for u in range(nslot):
        stage(g * nslot + u, u, traced_guard)
      return carry

    lax.fori_loop(0, ngroups, group, 0, unroll=False)
    for u in range(ntail):
      stage(ngroups * nslot + u, u, static_guard)
    plsc.subcore_barrier()

    pltpu.sync_copy(table_s.at[pl.ds(s * rpt, rpt)],
                    out_hbm.at[c, pl.ds(s * rpt, rpt)])
    if with_deg:
      pltpu.sync_copy(degtab_s.at[pl.ds(s * rpt, rpt)],
                      deg_hbm.at[c, pl.ds(s * rpt, rpt)])

  kern = functools.partial(
      pl.kernel,
      mesh=mesh,
      compiler_params=pltpu.CompilerParams(use_tc_tiling_on_sc=False),
      out_type=tuple(out_type) if with_deg else out_type[0],
      scratch_types=scratch,
  )

  @kern
  def seg(*refs):
    it = iter(refs)
    tab, ei, zrows = next(it), next(it), next(it)
    zdeg = next(it) if with_deg else None
    out = next(it)
    deg = next(it) if with_deg else None
    idx_v, rows_v, table_s = next(it), next(it), next(it)
    isems_s, isems_d, gsems = next(it), next(it), next(it)
    ones_v = next(it) if with_deg else None
    degtab_s = next(it) if with_deg else None
    tidx_v = next(it) if etail else None
    trows_v = next(it) if etail else None
    tsem = next(it) if etail else None
    body(tab, ei, zrows, zdeg, out, deg,
         idx_v, rows_v, table_s, isems_s, isems_d, gsems,
         ones_v, degtab_s, tidx_v, trows_v, tsem)

  return seg


def _layer1_body(feat_ref, degt_ref, x_ref, w1_ref, b1_ref, w2_ref,
                 p_ref, invd_ref):
  aggx = feat_ref[0] + feat_ref[1] + x_ref[...]
  deg16 = degt_ref[0] + degt_ref[1]
  inv = 1.0 / (jnp.max(deg16, axis=1, keepdims=True) + 1.0)
  h = aggx * inv
  h1 = jnp.maximum(
      jnp.dot(h, w1_ref[...], preferred_element_type=jnp.float32)
      + b1_ref[...], 0.0)
  p_ref[...] = jnp.dot(h1, w2_ref[...], preferred_element_type=jnp.float32)
  invd_ref[...] = jnp.broadcast_to(inv, invd_ref.shape)


def _layer2_body(agg_ref, p_ref, invd_ref, b2_ref, z_ref):
  z_ref[...] = ((agg_ref[0] + agg_ref[1] + p_ref[...]) * invd_ref[...]
                + b2_ref[...])


def _decoder_body(zr_ref, zc_ref, out_ref):
  out_ref[...] = lax.dot_general(
      zr_ref[...], zc_ref[...], (((1,), (1,)), ((), ())),
      preferred_element_type=jnp.float32)


def kernel(x, edge_index, W1, b1, W2, b2):
  n, din = x.shape
  e = edge_index.shape[1]
  h1_dim = W1.shape[1]
  h2 = W2.shape[1]

  rpt = (n // NS + 7) // 8 * 8
  z1 = jnp.zeros((rpt, din), jnp.float32)
  zd = jnp.zeros((rpt, DG), jnp.float32)
  z2 = jnp.zeros((rpt, h2), jnp.float32)

  seg1 = _make_seg_sum(n, e, din, 80, 3, with_deg=True)
  seg2 = _make_seg_sum(n, e, h2, 128, 3, with_deg=False)

  feat, degt = seg1(x, edge_index, z1, zd)

  rblk = 2000
  grid1 = n // rblk
  p, invd = pl.pallas_call(
      _layer1_body,
      grid=(grid1,),
      in_specs=[
          pl.BlockSpec((NC, rblk, din), lambda i: (0, i, 0)),
          pl.BlockSpec((NC, rblk, DG), lambda i: (0, i, 0)),
          pl.BlockSpec((rblk, din), lambda i: (i, 0)),
          pl.BlockSpec((din, h1_dim), lambda i: (0, 0)),
          pl.BlockSpec((1, h1_dim), lambda i: (0, 0)),
          pl.BlockSpec((h1_dim, h2), lambda i: (0, 0)),
      ],
      out_specs=[
          pl.BlockSpec((rblk, h2), lambda i: (i, 0)),
          pl.BlockSpec((rblk, h2), lambda i: (i, 0)),
      ],
      out_shape=[
          jax.ShapeDtypeStruct((n, h2), jnp.float32),
          jax.ShapeDtypeStruct((n, h2), jnp.float32),
      ],
  )(feat, degt, x, W1, b1.reshape(1, h1_dim), W2)

  agg2 = seg2(p, edge_index, z2)

  z = pl.pallas_call(
      _layer2_body,
      grid=(grid1,),
      in_specs=[
          pl.BlockSpec((NC, rblk, h2), lambda i: (0, i, 0)),
          pl.BlockSpec((rblk, h2), lambda i: (i, 0)),
          pl.BlockSpec((rblk, h2), lambda i: (i, 0)),
          pl.BlockSpec((1, h2), lambda i: (0, 0)),
      ],
      out_specs=pl.BlockSpec((rblk, h2), lambda i: (i, 0)),
      out_shape=jax.ShapeDtypeStruct((n, h2), jnp.float32),
  )(agg2, p, invd, b2.reshape(1, h2))

  ablk = 1024
  grid_a = pl.cdiv(n, ablk)
  adj = pl.pallas_call(
      _decoder_body,
      grid=(grid_a, grid_a),
      in_specs=[
          pl.BlockSpec((ablk, h2), lambda i, j: (i, 0)),
          pl.BlockSpec((ablk, h2), lambda i, j: (j, 0)),
      ],
      out_specs=pl.BlockSpec((ablk, ablk), lambda i, j: (i, j)),
      out_shape=jax.ShapeDtypeStruct((n, n), jnp.float32),
  )(z, z)

  return (z, adj)


# speedup vs baseline: 1.2192x; 1.1168x over previous
import functools

import jax
import jax.numpy as jnp
from jax import lax
from jax.experimental import pallas as pl
from jax.experimental.pallas import tpu as pltpu
from jax.experimental.pallas import tpu_sc as plsc

NC = 2
NS = 16
NW = NC * NS
DG = 16


def _make_seg_sum(n, e, d, ch, nbuf, with_deg):
  et = e // NW
  nchunks = et // ch
  etail = et % ch
  nslot = 2 * nbuf
  ngroups = nchunks // nslot
  ntail = nchunks % nslot
  assert ch % 8 == 0 and (nchunks * ch) % 8 == 0 and nchunks >= nslot
  rpt = (n // NS + 7) // 8 * 8
  npad = rpt * NS

  out_type = [jax.ShapeDtypeStruct((NC, npad, d), jnp.float32)]
  scratch = [
      pltpu.VMEM((nslot, 2, ch), jnp.int32),
      pltpu.VMEM((nbuf, ch, d), jnp.float32),
      pltpu.VMEM_SHARED((npad, d), jnp.float32),
      pltpu.SemaphoreType.DMA((nslot,)),
      pltpu.SemaphoreType.DMA((nslot,)),
      pltpu.SemaphoreType.DMA((nbuf,)),
  ]
  if with_deg:
    out_type.append(jax.ShapeDtypeStruct((NC, npad, DG), jnp.float32))
    scratch.append(pltpu.VMEM((ch, DG), jnp.float32))
    scratch.append(pltpu.VMEM_SHARED((npad, DG), jnp.float32))
  if etail:
    scratch.append(pltpu.VMEM((2, etail), jnp.int32))
    scratch.append(pltpu.VMEM((etail, d), jnp.float32))
    scratch.append(pltpu.SemaphoreType.DMA)

  mesh = plsc.VectorSubcoreMesh(core_axis_name="c", subcore_axis_name="s")

  def body(tab_hbm, ei_hbm, zrows_hbm, zdeg_hbm, out_hbm, deg_hbm,
           idx_v, rows_v, table_s, isems_s, isems_d, gsems,
           ones_v, degtab_s, tidx_v, trows_v, tsem):
    c = lax.axis_index("c")
    s = lax.axis_index("s")
    wid = s * NC + c
    ebase = wid * et

    pltpu.sync_copy(zrows_hbm, table_s.at[pl.ds(s * rpt, rpt)])
    if with_deg:
      pltpu.sync_copy(zdeg_hbm, degtab_s.at[pl.ds(s * rpt, rpt)])
      for i in range(ch):
        ones_v[i] = jnp.ones((DG,), jnp.float32)

    def load_idx(j, q):
      eb = ebase + j * ch
      pltpu.async_copy(ei_hbm.at[0, pl.ds(eb, ch)], idx_v.at[q, 0],
                       isems_s.at[q])
      pltpu.async_copy(ei_hbm.at[1, pl.ds(eb, ch)], idx_v.at[q, 1],
                       isems_d.at[q])

    def issue_gather(q, b):
      pltpu.make_async_copy(ei_hbm.at[0, pl.ds(0, ch)], idx_v.at[q, 0],
                            isems_s.at[q]).wait()
      pltpu.async_copy(tab_hbm.at[idx_v.at[q, 0]], rows_v.at[b], gsems.at[b])

    for q in range(nslot):
      load_idx(q, q)
    for b in range(nbuf):
      issue_gather(b, b)

    plsc.subcore_barrier()

    if etail:
      tb = ebase + nchunks * ch
      pltpu.sync_copy(ei_hbm.at[0, pl.ds(tb, etail)], tidx_v.at[0])
      pltpu.sync_copy(ei_hbm.at[1, pl.ds(tb, etail)], tidx_v.at[1])
      pltpu.async_copy(tab_hbm.at[tidx_v.at[0]], trows_v, tsem).wait()
      pltpu.sync_copy(trows_v, table_s.at[tidx_v.at[1]], add=True)
      if with_deg:
        pltpu.sync_copy(ones_v.at[pl.ds(0, etail)],
                        degtab_s.at[tidx_v.at[1]], add=True)

    def stage(j, u, guard):
      b = u % nbuf
      pltpu.make_async_copy(tab_hbm.at[pl.ds(0, ch)], rows_v.at[b],
                            gsems.at[b]).wait()
      pltpu.make_async_copy(ei_hbm.at[0, pl.ds(0, ch)], idx_v.at[u, 1],
                            isems_d.at[u]).wait()
      pltpu.sync_copy(rows_v.at[b], table_s.at[idx_v.at[u, 1]], add=True)
      if with_deg:
        pltpu.sync_copy(ones_v, degtab_s.at[idx_v.at[u, 1]], add=True)

      guard(j + nslot < nchunks, lambda: load_idx(j + nslot, u))
      qn = (u + nbuf) % nslot
      guard(j + nbuf < nchunks, lambda: issue_gather(qn, b))

    def traced_guard(cond, fn):
      pl.when(cond)(fn)

    def static_guard(cond, fn):
      if cond:
        fn()

    def group(g, carry):
      for u in range(nslot):
        stage(g * nslot + u, u, traced_guard)
      return carry

    lax.fori_loop(0, ngroups, group, 0, unroll=False)
    for u in range(ntail):
      stage(ngroups * nslot + u, u, static_guard)
    plsc.subcore_barrier()

    pltpu.sync_copy(table_s.at[pl.ds(s * rpt, rpt)],
                    out_hbm.at[c, pl.ds(s * rpt, rpt)])
    if with_deg:
      pltpu.sync_copy(degtab_s.at[pl.ds(s * rpt, rpt)],
                      deg_hbm.at[c, pl.ds(s * rpt, rpt)])

  kern = functools.partial(
      pl.kernel,
      mesh=mesh,
      compiler_params=pltpu.CompilerParams(use_tc_tiling_on_sc=False),
      out_type=tuple(out_type) if with_deg else out_type[0],
      scratch_types=scratch,
  )

  @kern
  def seg(*refs):
    it = iter(refs)
    tab, ei, zrows = next(it), next(it), next(it)
    zdeg = next(it) if with_deg else None
    out = next(it)
    deg = next(it) if with_deg else None
    idx_v, rows_v, table_s = next(it), next(it), next(it)
    isems_s, isems_d, gsems = next(it), next(it), next(it)
    ones_v = next(it) if with_deg else None
    degtab_s = next(it) if with_deg else None
    tidx_v = next(it) if etail else None
    trows_v = next(it) if etail else None
    tsem = next(it) if etail else None
    body(tab, ei, zrows, zdeg, out, deg,
         idx_v, rows_v, table_s, isems_s, isems_d, gsems,
         ones_v, degtab_s, tidx_v, trows_v, tsem)

  return seg


def _layer1_body(feat_ref, degt_ref, x_ref, w1_ref, b1_ref, w2_ref,
                 p_ref, invd_ref):
  aggx = feat_ref[0] + feat_ref[1] + x_ref[...]
  deg16 = degt_ref[0] + degt_ref[1]
  inv = 1.0 / (jnp.max(deg16, axis=1, keepdims=True) + 1.0)
  h = aggx * inv
  h1 = jnp.maximum(
      jnp.dot(h, w1_ref[...], preferred_element_type=jnp.float32)
      + b1_ref[...], 0.0)
  p_ref[...] = jnp.dot(h1, w2_ref[...], preferred_element_type=jnp.float32)
  invd_ref[...] = jnp.broadcast_to(inv, invd_ref.shape)


def _layer2_body(agg_ref, p_ref, invd_ref, b2_ref, z_ref):
  z_ref[...] = ((agg_ref[0] + agg_ref[1] + p_ref[...]) * invd_ref[...]
                + b2_ref[...])


def _decoder_body(zr_ref, zc_ref, out_ref):
  out_ref[...] = lax.dot_general(
      zr_ref[...], zc_ref[...], (((1,), (1,)), ((), ())),
      preferred_element_type=jnp.float32)


def kernel(x, edge_index, W1, b1, W2, b2):
  n, din = x.shape
  e = edge_index.shape[1]
  h1_dim = W1.shape[1]
  h2 = W2.shape[1]

  rpt = (n // NS + 7) // 8 * 8
  z1 = jnp.zeros((rpt, din), jnp.float32)
  zd = jnp.zeros((rpt, DG), jnp.float32)
  z2 = jnp.zeros((rpt, h2), jnp.float32)

  seg1 = _make_seg_sum(n, e, din, 80, 3, with_deg=True)
  seg2 = _make_seg_sum(n, e, h2, 128, 3, with_deg=False)

  feat, degt = seg1(x, edge_index, z1, zd)

  rblk = 2000
  grid1 = n // rblk
  p, invd = pl.pallas_call(
      _layer1_body,
      grid=(grid1,),
      in_specs=[
          pl.BlockSpec((NC, rblk, din), lambda i: (0, i, 0)),
          pl.BlockSpec((NC, rblk, DG), lambda i: (0, i, 0)),
          pl.BlockSpec((rblk, din), lambda i: (i, 0)),
          pl.BlockSpec((din, h1_dim), lambda i: (0, 0)),
          pl.BlockSpec((1, h1_dim), lambda i: (0, 0)),
          pl.BlockSpec((h1_dim, h2), lambda i: (0, 0)),
      ],
      out_specs=[
          pl.BlockSpec((rblk, h2), lambda i: (i, 0)),
          pl.BlockSpec((rblk, h2), lambda i: (i, 0)),
      ],
      out_shape=[
          jax.ShapeDtypeStruct((n, h2), jnp.float32),
          jax.ShapeDtypeStruct((n, h2), jnp.float32),
      ],
  )(feat, degt, x, W1, b1.reshape(1, h1_dim), W2)

  agg2 = seg2(p, edge_index, z2)

  z = pl.pallas_call(
      _layer2_body,
      grid=(grid1,),
      in_specs=[
          pl.BlockSpec((NC, rblk, h2), lambda i: (0, i, 0)),
          pl.BlockSpec((rblk, h2), lambda i: (i, 0)),
          pl.BlockSpec((rblk, h2), lambda i: (i, 0)),
          pl.BlockSpec((1, h2), lambda i: (0, 0)),
      ],
      out_specs=pl.BlockSpec((rblk, h2), lambda i: (i, 0)),
      out_shape=jax.ShapeDtypeStruct((n, h2), jnp.float32),
  )(agg2, p, invd, b2.reshape(1, h2))

  ablk = 512
  grid_a = pl.cdiv(n, ablk)
  adj = pl.pallas_call(
      _decoder_body,
      grid=(grid_a,),
      in_specs=[
          pl.BlockSpec((ablk, h2), lambda i: (i, 0)),
          pl.BlockSpec((n, h2), lambda i: (0, 0)),
      ],
      out_specs=pl.BlockSpec((ablk, n), lambda i: (i, 0)),
      out_shape=jax.ShapeDtypeStruct((n, n), jnp.float32),
      compiler_params=pltpu.CompilerParams(
          vmem_limit_bytes=100 * 1024 * 1024),
  )(z, z)

  return (z, adj)
